# baseline (speedup 1.0000x reference)
import jax
import jax.numpy as jnp
from jax.experimental import pallas as pl
from jax.experimental.pallas import tpu as pltpu

_BLOCK_N = 8192


def _contract_kernel(x_ref, e_ref, o_ref):
    o_ref[...] = jax.lax.dot_general(
        x_ref[...],
        e_ref[...],
        dimension_numbers=(((1,), (0,)), ((), ())),
        preferred_element_type=jnp.float32,
        precision=jax.lax.Precision.HIGHEST,
    )


def kernel(inputs, channel_embeddings):
    B, H, W, C = inputs.shape
    D = channel_embeddings.shape[1]
    N = B * H * W
    x = inputs.reshape(N, C)

    num_blocks = pl.cdiv(N, _BLOCK_N)
    out = pl.pallas_call(
        _contract_kernel,
        grid=(num_blocks,),
        in_specs=[
            pl.BlockSpec((_BLOCK_N, C), lambda i: (i, 0)),
            pl.BlockSpec((C, D), lambda i: (0, 0)),
        ],
        out_specs=pl.BlockSpec((_BLOCK_N, D), lambda i: (i, 0)),
        out_shape=jax.ShapeDtypeStruct((N, D), jnp.float32),
        compiler_params=pltpu.CompilerParams(
            dimension_semantics=("arbitrary",),
        ),
    )(x, channel_embeddings)
    return out.reshape(B, H, W, D)


# speedup vs baseline: 1.0581x; 1.0581x over previous
import jax
import jax.numpy as jnp
from jax.experimental import pallas as pl
from jax.experimental.pallas import tpu as pltpu

_BLOCK_N = 8192


def _contract_kernel(x_ref, e_ref, o_ref):
    o_ref[...] = jax.lax.dot_general(
        x_ref[...].astype(jnp.bfloat16),
        e_ref[...].astype(jnp.bfloat16),
        dimension_numbers=(((1,), (0,)), ((), ())),
        preferred_element_type=jnp.float32,
    )


def kernel(inputs, channel_embeddings):
    B, H, W, C = inputs.shape
    D = channel_embeddings.shape[1]
    N = B * H * W
    x = inputs.reshape(N, C)

    num_blocks = pl.cdiv(N, _BLOCK_N)
    out = pl.pallas_call(
        _contract_kernel,
        grid=(num_blocks,),
        in_specs=[
            pl.BlockSpec((_BLOCK_N, C), lambda i: (i, 0)),
            pl.BlockSpec((C, D), lambda i: (0, 0)),
        ],
        out_specs=pl.BlockSpec((_BLOCK_N, D), lambda i: (i, 0)),
        out_shape=jax.ShapeDtypeStruct((N, D), jnp.float32),
        compiler_params=pltpu.CompilerParams(
            dimension_semantics=("arbitrary",),
        ),
    )(x, channel_embeddings)
    return out.reshape(B, H, W, D)


# speedup vs baseline: 2.2077x; 2.0865x over previous
import jax
import jax.numpy as jnp
from jax.experimental import pallas as pl
from jax.experimental.pallas import tpu as pltpu

_BLOCK_H = 56


def _contract_kernel(x_ref, e_ref, o_ref):
    bh, h, w, c = x_ref.shape
    d = e_ref.shape[1]
    x = x_ref[...].reshape(bh * h * w, c)
    out = jax.lax.dot_general(
        x.astype(jnp.bfloat16),
        e_ref[...].astype(jnp.bfloat16),
        dimension_numbers=(((1,), (0,)), ((), ())),
        preferred_element_type=jnp.float32,
    )
    o_ref[...] = out.reshape(bh, h, w, d)


def kernel(inputs, channel_embeddings):
    B, H, W, C = inputs.shape
    D = channel_embeddings.shape[1]

    out = pl.pallas_call(
        _contract_kernel,
        grid=(B, H // _BLOCK_H),
        in_specs=[
            pl.BlockSpec((1, _BLOCK_H, W, C), lambda b, h: (b, h, 0, 0)),
            pl.BlockSpec((C, D), lambda b, h: (0, 0)),
        ],
        out_specs=pl.BlockSpec((1, _BLOCK_H, W, D), lambda b, h: (b, h, 0, 0)),
        out_shape=jax.ShapeDtypeStruct((B, H, W, D), jnp.float32),
        compiler_params=pltpu.CompilerParams(
            dimension_semantics=("arbitrary", "arbitrary"),
        ),
    )(inputs, channel_embeddings)
    return out


# speedup vs baseline: 14.0367x; 6.3582x over previous
import jax
import jax.numpy as jnp
from jax.experimental import pallas as pl
from jax.experimental.pallas import tpu as pltpu

_BLOCK_H = 56


def _contract_kernel(x_ref, e_ref, o_ref):
    e = e_ref[...]
    for h in range(x_ref.shape[1]):
        x = x_ref[0, h].astype(jnp.bfloat16)
        o_ref[0, h] = jax.lax.dot_general(
            e,
            x,
            dimension_numbers=(((1,), (0,)), ((), ())),
            preferred_element_type=jnp.float32,
        )


def kernel(inputs, channel_embeddings):
    B, H, W, C = inputs.shape
    D = channel_embeddings.shape[1]

    x_t = jnp.transpose(inputs, (0, 1, 3, 2))
    e_t = jnp.transpose(channel_embeddings, (1, 0)).astype(jnp.bfloat16)

    out_t = pl.pallas_call(
        _contract_kernel,
        grid=(B, H // _BLOCK_H),
        in_specs=[
            pl.BlockSpec((1, _BLOCK_H, C, W), lambda b, h: (b, h, 0, 0)),
            pl.BlockSpec((D, C), lambda b, h: (0, 0)),
        ],
        out_specs=pl.BlockSpec((1, _BLOCK_H, D, W), lambda b, h: (b, h, 0, 0)),
        out_shape=jax.ShapeDtypeStruct((B, H, D, W), jnp.float32),
        compiler_params=pltpu.CompilerParams(
            dimension_semantics=("arbitrary", "arbitrary"),
        ),
    )(x_t, e_t)
    return jnp.transpose(out_t, (0, 1, 3, 2))


# speedup vs baseline: 14.2024x; 1.0118x over previous
import jax
import jax.numpy as jnp
from jax.experimental import pallas as pl
from jax.experimental.pallas import tpu as pltpu

_BLOCK_H = 112


def _contract_kernel(x_ref, e_ref, o_ref):
    e = e_ref[...]
    for h in range(x_ref.shape[1]):
        x = x_ref[0, h].astype(jnp.bfloat16)
        o_ref[0, h] = jax.lax.dot_general(
            e,
            x,
            dimension_numbers=(((1,), (0,)), ((), ())),
            preferred_element_type=jnp.float32,
        )


def kernel(inputs, channel_embeddings):
    B, H, W, C = inputs.shape
    D = channel_embeddings.shape[1]

    x_t = jnp.transpose(inputs, (0, 1, 3, 2))
    e_t = jnp.transpose(channel_embeddings, (1, 0)).astype(jnp.bfloat16)

    out_t = pl.pallas_call(
        _contract_kernel,
        grid=(B, H // _BLOCK_H),
        in_specs=[
            pl.BlockSpec((1, _BLOCK_H, C, W), lambda b, h: (b, h, 0, 0)),
            pl.BlockSpec((D, C), lambda b, h: (0, 0)),
        ],
        out_specs=pl.BlockSpec((1, _BLOCK_H, D, W), lambda b, h: (b, h, 0, 0)),
        out_shape=jax.ShapeDtypeStruct((B, H, D, W), jnp.float32),
        compiler_params=pltpu.CompilerParams(
            dimension_semantics=("arbitrary", "arbitrary"),
        ),
    )(x_t, e_t)
    return jnp.transpose(out_t, (0, 1, 3, 2))
